# baseline (speedup 1.0000x reference)
import functools

import jax
import jax.numpy as jnp
from jax import lax
from jax.experimental import pallas as pl
from jax.experimental.pallas import tpu as pltpu
from jax.experimental.pallas import tpu_sc as plsc


def _build(B, L, SL, N):
    NC, NS = 2, 16
    NW = NC * NS
    assert B % NW == 0
    rows_per_worker = B // NW
    RB = 16
    assert rows_per_worker % RB == 0
    chunks = rows_per_worker // RB
    WPR = L // 4
    JV = SL // 16

    mesh = plsc.VectorSubcoreMesh(core_axis_name="c", subcore_axis_name="s")

    @functools.partial(
        pl.kernel,
        out_type=jax.ShapeDtypeStruct((N, B, SL), jnp.float32),
        mesh=mesh,
        scratch_types=[
            pltpu.VMEM((RB, WPR), jnp.int32),
            pltpu.VMEM((RB, SL), jnp.float32),
            pltpu.VMEM((N,), jnp.int32),
            pltpu.VMEM((RB, SL), jnp.float32),
        ],
        compiler_params=pltpu.CompilerParams(needs_layout_passes=False),
    )
    def k(w_hbm, s_hbm, idx_hbm, out_hbm, wrows, srows, svmem, obuf):
        wid = lax.axis_index("s") * NC + lax.axis_index("c")
        pltpu.sync_copy(idx_hbm, svmem)
        lane = lax.iota(jnp.int32, 16)

        def chunk_body(c, _):
            base = wid * rows_per_worker + c * RB
            pltpu.sync_copy(w_hbm.at[pl.ds(base, RB)], wrows)
            pltpu.sync_copy(s_hbm.at[pl.ds(base, RB)], srows)

            def i_body(i, _):
                s_vec = plsc.load_gather(
                    svmem, [jnp.zeros((16,), jnp.int32) + i])

                def j_body(j, _):
                    pos = s_vec + j * 16 + lane
                    widx = pos >> 2
                    shift = (pos & 3) << 3
                    for bl in range(RB):
                        w = plsc.load_gather(
                            wrows, [jnp.full((16,), bl, jnp.int32), widx])
                        m = ((w >> shift) & 0xFF) != 0
                        vals = srows[bl, pl.ds(j * 16, 16)]
                        obuf[bl, pl.ds(j * 16, 16)] = jnp.where(m, vals, 0.0)
                    return 0

                lax.fori_loop(0, JV, j_body, 0)
                pltpu.sync_copy(obuf, out_hbm.at[i, pl.ds(base, RB)])
                return 0

            lax.fori_loop(0, N, i_body, 0)
            return 0

        lax.fori_loop(0, chunks, chunk_body, 0)

    return k


def kernel(where_input, slice_input, slice_len, start_indices):
    B, L = where_input.shape
    SL = slice_input.shape[1]
    N = start_indices.shape[0]
    w_i32 = where_input.view(jnp.uint8).view(jnp.int32)
    zero_off = (jnp.asarray(slice_len) - SL).astype(jnp.int32)
    starts = jnp.clip(
        start_indices.astype(jnp.int32) + zero_off, 0, L - SL)
    return _build(B, L, SL, N)(w_i32, slice_input, starts)


# speedup vs baseline: 4.0659x; 4.0659x over previous
import functools

import jax
import jax.numpy as jnp
from jax import lax
from jax.experimental import pallas as pl
from jax.experimental.pallas import tpu as pltpu
from jax.experimental.pallas import tpu_sc as plsc


def _build(B, L, SL, N):
    NC, NS = 2, 16
    NW = NC * NS
    assert B % NW == 0
    rows_per_worker = B // NW
    RB = 16
    assert rows_per_worker % RB == 0
    chunks = rows_per_worker // RB
    WPR = L // 4
    JV = SL // 16
    HP = N // 2

    mesh = plsc.VectorSubcoreMesh(core_axis_name="c", subcore_axis_name="s")

    @functools.partial(
        pl.kernel,
        out_type=jax.ShapeDtypeStruct((N, B, SL), jnp.float32),
        mesh=mesh,
        scratch_types=[
            pltpu.VMEM((RB, L // 32), jnp.int32),
            pltpu.VMEM((RB, SL), jnp.float32),
            pltpu.VMEM((N,), jnp.int32),
            pltpu.VMEM((RB, SL), jnp.float32),
            pltpu.VMEM((RB, SL), jnp.float32),
            pltpu.SemaphoreType.DMA,
            pltpu.SemaphoreType.DMA,
        ],
        compiler_params=pltpu.CompilerParams(needs_layout_passes=False),
    )
    def k(w_hbm, s_hbm, idx_hbm, out_hbm, wrows, srows, svmem,
          ob0, ob1, sem0, sem1):
        wid = lax.axis_index("s") * NC + lax.axis_index("c")
        pltpu.sync_copy(idx_hbm, svmem)
        lane = lax.iota(jnp.int32, 16)
        zeros16 = jnp.zeros((16,), jnp.int32)

        def compute_i(i, ob):
            s_vec = plsc.load_gather(svmem, [zeros16 + i])

            @plsc.parallel_loop(0, JV)
            def _(j):
                pos = s_vec + j * 16 + lane
                widx = pos >> 5
                bmask = jnp.int32(1) << (pos & 31)
                for bl in range(RB):
                    w = plsc.load_gather(wrows, [zeros16 + bl, widx])
                    m = (w & bmask) != 0
                    ob[bl, pl.ds(j * 16, 16)] = jnp.where(
                        m, srows[bl, pl.ds(j * 16, 16)], 0.0)

        def t_body(t, _):
            c = t // HP
            ip = t % HP
            base = wid * rows_per_worker + c * RB

            @pl.when(ip == 0)
            def _():
                pltpu.sync_copy(w_hbm.at[pl.ds(base, RB)], wrows)
                pltpu.sync_copy(s_hbm.at[pl.ds(base, RB)], srows)

            @pl.when(t > 0)
            def _():
                pltpu.make_async_copy(
                    ob0, out_hbm.at[0, pl.ds(0, RB)], sem0).wait()

            compute_i(2 * ip, ob0)
            pltpu.async_copy(ob0, out_hbm.at[2 * ip, pl.ds(base, RB)], sem0)

            @pl.when(t > 0)
            def _():
                pltpu.make_async_copy(
                    ob1, out_hbm.at[0, pl.ds(0, RB)], sem1).wait()

            compute_i(2 * ip + 1, ob1)
            pltpu.async_copy(
                ob1, out_hbm.at[2 * ip + 1, pl.ds(base, RB)], sem1)
            return 0

        lax.fori_loop(0, chunks * HP, t_body, 0)
        pltpu.make_async_copy(ob0, out_hbm.at[0, pl.ds(0, RB)], sem0).wait()
        pltpu.make_async_copy(ob1, out_hbm.at[0, pl.ds(0, RB)], sem1).wait()

    return k


def kernel(where_input, slice_input, slice_len, start_indices):
    B, L = where_input.shape
    SL = slice_input.shape[1]
    N = start_indices.shape[0]
    zero_off = (jnp.asarray(slice_len) - SL).astype(jnp.int32)
    starts = jnp.clip(
        start_indices.astype(jnp.int32) + zero_off, 0, L - SL)
    wbits = (where_input.reshape(B, L // 32, 32).astype(jnp.uint32)
             << jnp.arange(32, dtype=jnp.uint32)).sum(
                 axis=-1, dtype=jnp.uint32).astype(jnp.int32)
    return _build(B, L, SL, N)(wbits, slice_input, starts)


# speedup vs baseline: 6.1674x; 1.5169x over previous
import functools

import jax
import jax.numpy as jnp
from jax import lax
from jax.experimental import pallas as pl
from jax.experimental.pallas import tpu as pltpu
from jax.experimental.pallas import tpu_sc as plsc


def _build(B, L, SL, N):
    NC, NS = 2, 16
    NW = NC * NS
    assert B % NW == 0
    rows_per_worker = B // NW
    RB = 16
    assert rows_per_worker % RB == 0
    chunks = rows_per_worker // RB
    WB = L // 32
    WSH = (L // 32).bit_length() - 1
    JV = SL // 16
    HP = N // 2

    mesh = plsc.VectorSubcoreMesh(core_axis_name="c", subcore_axis_name="s")

    @functools.partial(
        pl.kernel,
        out_type=jax.ShapeDtypeStruct((N, B, SL), jnp.float32),
        mesh=mesh,
        scratch_types=[
            pltpu.VMEM((RB, L // 32), jnp.int32),
            pltpu.VMEM((RB, SL), jnp.float32),
            pltpu.VMEM((N,), jnp.int32),
            pltpu.VMEM((RB, SL), jnp.float32),
            pltpu.VMEM((RB, SL), jnp.float32),
            pltpu.SemaphoreType.DMA,
            pltpu.SemaphoreType.DMA,
        ],
        compiler_params=pltpu.CompilerParams(needs_layout_passes=False),
    )
    def k(w_hbm, s_hbm, idx_hbm, out_hbm, wrows, srows, svmem,
          ob0, ob1, sem0, sem1):
        wid = lax.axis_index("s") * NC + lax.axis_index("c")
        pltpu.sync_copy(idx_hbm, svmem)
        lane = lax.iota(jnp.int32, 16)
        zeros16 = jnp.zeros((16,), jnp.int32)

        def compute_i(i, ob):
            s_vec = plsc.load_gather(svmem, [zeros16 + i])

            @plsc.parallel_loop(0, JV)
            def _(j):
                pos = s_vec + j * 16 + lane
                widx = pos & (WB - 1)
                bmask = jnp.int32(1) << (pos >> WSH)
                for bl in range(RB):
                    w = plsc.load_gather(wrows, [zeros16 + bl, widx])
                    m = (w & bmask) != 0
                    ob[bl, pl.ds(j * 16, 16)] = jnp.where(
                        m, srows[bl, pl.ds(j * 16, 16)], 0.0)

        def t_body(t, _):
            c = t // HP
            ip = t % HP
            base = wid * rows_per_worker + c * RB

            @pl.when(ip == 0)
            def _():
                pltpu.sync_copy(w_hbm.at[pl.ds(base, RB)], wrows)
                pltpu.sync_copy(s_hbm.at[pl.ds(base, RB)], srows)

            @pl.when(t > 0)
            def _():
                pltpu.make_async_copy(
                    ob0, out_hbm.at[0, pl.ds(0, RB)], sem0).wait()

            compute_i(2 * ip, ob0)
            pltpu.async_copy(ob0, out_hbm.at[2 * ip, pl.ds(base, RB)], sem0)

            @pl.when(t > 0)
            def _():
                pltpu.make_async_copy(
                    ob1, out_hbm.at[0, pl.ds(0, RB)], sem1).wait()

            compute_i(2 * ip + 1, ob1)
            pltpu.async_copy(
                ob1, out_hbm.at[2 * ip + 1, pl.ds(base, RB)], sem1)
            return 0

        lax.fori_loop(0, chunks * HP, t_body, 0)
        pltpu.make_async_copy(ob0, out_hbm.at[0, pl.ds(0, RB)], sem0).wait()
        pltpu.make_async_copy(ob1, out_hbm.at[0, pl.ds(0, RB)], sem1).wait()

    return k


def kernel(where_input, slice_input, slice_len, start_indices):
    B, L = where_input.shape
    SL = slice_input.shape[1]
    N = start_indices.shape[0]
    zero_off = (jnp.asarray(slice_len) - SL).astype(jnp.int32)
    starts = jnp.clip(
        start_indices.astype(jnp.int32) + zero_off, 0, L - SL)
    WB = L // 32
    wbits = jnp.where(where_input[:, :WB], jnp.int32(1), jnp.int32(0))
    for kk in range(1, 32):
        wbits = wbits | jnp.where(
            where_input[:, kk * WB:(kk + 1) * WB], jnp.int32(1) << kk,
            jnp.int32(0))
    return _build(B, L, SL, N)(wbits, slice_input, starts)
